# baseline (speedup 1.0000x reference)
import functools

import jax
import jax.numpy as jnp
from jax import lax
from jax.experimental import pallas as pl
from jax.experimental.pallas import tpu as pltpu

_N = 16384
_NP = 1024
_K = 32
_R = 128
_C = 128


def _fps_body(px_ref, py_ref, pz_ref, out_ref, dists_ref):
    b = pl.program_id(0)
    px = px_ref[0]
    py = py_ref[0]
    pz = pz_ref[0]
    dists_ref[...] = jnp.full((_R, _C), jnp.inf, jnp.float32)
    out_ref[b, 0] = jnp.int32(0)

    lane_iota = lax.broadcasted_iota(jnp.int32, (1, _C), 1)
    row_iota = lax.broadcasted_iota(jnp.int32, (_R, _C), 0)
    col_iota = lax.broadcasted_iota(jnp.int32, (_R, _C), 1)
    lin_iota = row_iota * _C + col_iota

    def body(i, last):
        r = last // _C
        c = last % _C
        cm = lane_iota == c
        qx = jnp.sum(jnp.where(cm, px_ref[0, pl.ds(r, 1), :], 0.0))
        qy = jnp.sum(jnp.where(cm, py_ref[0, pl.ds(r, 1), :], 0.0))
        qz = jnp.sum(jnp.where(cm, pz_ref[0, pl.ds(r, 1), :], 0.0))
        dx = px - qx
        dy = py - qy
        dz = pz - qz
        d = dx * dx + dy * dy + dz * dz
        dd = jnp.minimum(dists_ref[...], d)
        dists_ref[...] = dd
        m = jnp.max(dd)
        nxt = jnp.min(jnp.where(dd == m, lin_iota, jnp.int32(2**30)))
        out_ref[b, i] = nxt
        return nxt

    lax.fori_loop(1, _NP, body, jnp.int32(0))


def _fps_pallas(points):
    px = points[:, :, 0].reshape(2, _R, _C)
    py = points[:, :, 1].reshape(2, _R, _C)
    pz = points[:, :, 2].reshape(2, _R, _C)
    return pl.pallas_call(
        _fps_body,
        grid=(2,),
        in_specs=[pl.BlockSpec((1, _R, _C), lambda b: (b, 0, 0))] * 3,
        out_specs=pl.BlockSpec(memory_space=pltpu.SMEM),
        out_shape=jax.ShapeDtypeStruct((2, _NP), jnp.int32),
        scratch_shapes=[pltpu.VMEM((_R, _C), jnp.float32)],
    )(px, py, pz)


def _dist_body(q_ref, ct_ref, o_ref):
    q = q_ref[0]
    qx = q[:, 0:1]
    qy = q[:, 1:2]
    qz = q[:, 2:3]
    cx = ct_ref[0, 0:1, :]
    cy = ct_ref[0, 1:2, :]
    cz = ct_ref[0, 2:3, :]
    qq = qx * qx + qy * qy + qz * qz
    cc = cx * cx + cy * cy + cz * cz
    f32 = jnp.float32
    bf = jnp.bfloat16
    qxb, qyb, qzb = (v.astype(bf).astype(f32) for v in (qx, qy, qz))
    cxb, cyb, czb = (v.astype(bf).astype(f32) for v in (cx, cy, cz))
    p0 = qxb * cxb
    p1 = qyb * cyb
    p2 = qzb * czb
    s = p0 + p1
    bb = s - p0
    e1 = (p0 - (s - bb)) + (p1 - bb)
    t = s + p2
    bb2 = t - s
    e2 = (s - (t - bb2)) + (p2 - bb2)
    e = t + (e1 + e2)
    d = (cc + qq) - 2.0 * e
    o_ref[0] = jnp.maximum(d, 0.0)


def _dist_pallas(q, ct, nq_blk, nc_blk):
    B, M, _ = q.shape
    N = ct.shape[2]
    grid = (B, M // nq_blk, N // nc_blk)
    return pl.pallas_call(
        _dist_body,
        grid=grid,
        in_specs=[
            pl.BlockSpec((1, nq_blk, 3), lambda b, j, i: (b, j, 0)),
            pl.BlockSpec((1, 3, nc_blk), lambda b, j, i: (b, 0, i)),
        ],
        out_specs=pl.BlockSpec((1, nq_blk, nc_blk), lambda b, j, i: (b, j, i)),
        out_shape=jax.ShapeDtypeStruct((B, M, N), jnp.float32),
    )(q, ct)


def _pdist2squared(x, y):
    xx = jnp.sum(x ** 2, axis=1)[:, :, None]
    yy = jnp.sum(y ** 2, axis=1)[:, None, :]
    dist = xx + yy - 2.0 * jnp.einsum('bdn,bdm->bnm', x, y)
    dist = jnp.nan_to_num(dist, nan=0.0)
    return jnp.clip(dist, 0.0, jnp.inf)


def _knn_ind(xyz2, xyz1, k):
    dist = _pdist2squared(jnp.transpose(xyz2, (0, 2, 1)),
                          jnp.transpose(xyz1, (0, 2, 1)))
    dist_t = jnp.transpose(dist, (0, 2, 1))
    _, idx = jax.lax.top_k(-dist_t, k + 1)
    return idx[:, :, 1:]


def kernel(points):
    B, N, _ = points.shape
    xyz_ind = _fps_pallas(points)
    xyz_query = jax.vmap(lambda p, i: p[i])(points, xyz_ind)

    pts_t = jnp.transpose(points, (0, 2, 1))
    q_t = jnp.transpose(xyz_query, (0, 2, 1))
    dist_mid = _dist_pallas(xyz_query, pts_t, 128, 2048)
    _, idx = jax.lax.top_k(-dist_mid, _K + 1)
    neighbors_mid = idx[:, :, 1:]
    src_mid = neighbors_mid.reshape(B, -1)
    dst_mid = jnp.repeat(xyz_ind, _K, axis=1)
    d_mid = jax.vmap(lambda p, d, s: p[d] - p[s])(points, dst_mid, src_mid)

    dist_out = _dist_pallas(xyz_query, q_t, 128, 1024)
    _, idx2 = jax.lax.top_k(-dist_out, _K + 1)
    neighbors_out = idx2[:, :, 1:]
    src_out = neighbors_out.reshape(B, -1)
    dst_out = jnp.tile(jnp.repeat(jnp.arange(_NP, dtype=jnp.int32), _K)[None, :], (B, 1))
    d_out = jax.vmap(lambda p, d, s: p[d] - p[s])(xyz_query, dst_out, src_out)

    return (xyz_query, d_mid, d_out, xyz_ind, neighbors_mid, neighbors_out)


# speedup vs baseline: 2.0723x; 2.0723x over previous
import functools

import jax
import jax.numpy as jnp
from jax import lax
from jax.experimental import pallas as pl
from jax.experimental.pallas import tpu as pltpu

_N = 16384
_NP = 1024
_K = 32
_R = 128
_C = 128


def _fps_body(px_ref, py_ref, pz_ref, out_ref, dists_ref):
    b = pl.program_id(0)
    px = px_ref[0]
    py = py_ref[0]
    pz = pz_ref[0]
    dists_ref[...] = jnp.full((_R, _C), jnp.inf, jnp.float32)
    out_ref[b, 0] = jnp.int32(0)

    lane_iota = lax.broadcasted_iota(jnp.int32, (1, _C), 1)
    row_iota = lax.broadcasted_iota(jnp.int32, (_R, _C), 0)
    col_iota = lax.broadcasted_iota(jnp.int32, (_R, _C), 1)
    lin_iota = row_iota * _C + col_iota

    def body(i, last):
        r = last // _C
        c = last % _C
        cm = lane_iota == c
        qx = jnp.sum(jnp.where(cm, px_ref[0, pl.ds(r, 1), :], 0.0))
        qy = jnp.sum(jnp.where(cm, py_ref[0, pl.ds(r, 1), :], 0.0))
        qz = jnp.sum(jnp.where(cm, pz_ref[0, pl.ds(r, 1), :], 0.0))
        dx = px - qx
        dy = py - qy
        dz = pz - qz
        d = dx * dx + dy * dy + dz * dz
        dd = jnp.minimum(dists_ref[...], d)
        dists_ref[...] = dd
        m = jnp.max(dd)
        nxt = jnp.min(jnp.where(dd == m, lin_iota, jnp.int32(2**30)))
        out_ref[b, i] = nxt
        return nxt

    lax.fori_loop(1, _NP, body, jnp.int32(0))


def _fps_pallas(points):
    px = points[:, :, 0].reshape(2, _R, _C)
    py = points[:, :, 1].reshape(2, _R, _C)
    pz = points[:, :, 2].reshape(2, _R, _C)
    return pl.pallas_call(
        _fps_body,
        grid=(2,),
        in_specs=[pl.BlockSpec((1, _R, _C), lambda b: (b, 0, 0))] * 3,
        out_specs=pl.BlockSpec(memory_space=pltpu.SMEM),
        out_shape=jax.ShapeDtypeStruct((2, _NP), jnp.int32),
        scratch_shapes=[pltpu.VMEM((_R, _C), jnp.float32)],
    )(px, py, pz)


def _dist_body(q_ref, ct_ref, o_ref):
    q = q_ref[0]
    qx = q[:, 0:1]
    qy = q[:, 1:2]
    qz = q[:, 2:3]
    cx = ct_ref[0, 0:1, :]
    cy = ct_ref[0, 1:2, :]
    cz = ct_ref[0, 2:3, :]
    qq = qx * qx + qy * qy + qz * qz
    cc = cx * cx + cy * cy + cz * cz
    f32 = jnp.float32
    bf = jnp.bfloat16
    qxb, qyb, qzb = (v.astype(bf).astype(f32) for v in (qx, qy, qz))
    cxb, cyb, czb = (v.astype(bf).astype(f32) for v in (cx, cy, cz))
    p0 = qxb * cxb
    p1 = qyb * cyb
    p2 = qzb * czb
    s = p0 + p1
    bb = s - p0
    e1 = (p0 - (s - bb)) + (p1 - bb)
    t = s + p2
    bb2 = t - s
    e2 = (s - (t - bb2)) + (p2 - bb2)
    e = t + (e1 + e2)
    d = (cc + qq) - 2.0 * e
    o_ref[0] = jnp.maximum(d, 0.0)


def _dist_pallas(q, ct, nq_blk, nc_blk):
    B, M, _ = q.shape
    N = ct.shape[2]
    grid = (B, M // nq_blk, N // nc_blk)
    return pl.pallas_call(
        _dist_body,
        grid=grid,
        in_specs=[
            pl.BlockSpec((1, nq_blk, 3), lambda b, j, i: (b, j, 0)),
            pl.BlockSpec((1, 3, nc_blk), lambda b, j, i: (b, 0, i)),
        ],
        out_specs=pl.BlockSpec((1, nq_blk, nc_blk), lambda b, j, i: (b, j, i)),
        out_shape=jax.ShapeDtypeStruct((B, M, N), jnp.float32),
    )(q, ct)


def _topk_body(n, nchunk, q_ref, c_ref, out_ref, d_ref):
    f32 = jnp.float32
    bf = jnp.bfloat16
    qx = q_ref[0, 0:1, :]
    qy = q_ref[0, 1:2, :]
    qz = q_ref[0, 2:3, :]
    qq = qx * qx + qy * qy + qz * qz
    qxb, qyb, qzb = (v.astype(bf).astype(f32) for v in (qx, qy, qz))
    for i in range(n // nchunk):
        sl = pl.ds(i * nchunk, nchunk)
        cx = c_ref[0, sl, 0:1]
        cy = c_ref[0, sl, 1:2]
        cz = c_ref[0, sl, 2:3]
        cc = cx * cx + cy * cy + cz * cz
        cxb, cyb, czb = (v.astype(bf).astype(f32) for v in (cx, cy, cz))
        p0 = cxb * qxb
        p1 = cyb * qyb
        p2 = czb * qzb
        s = p0 + p1
        bb = s - p0
        e1 = (p0 - (s - bb)) + (p1 - bb)
        t = s + p2
        bb2 = t - s
        e2 = (s - (t - bb2)) + (p2 - bb2)
        e = t + (e1 + e2)
        d_ref[sl, :] = jnp.maximum((cc + qq) - 2.0 * e, 0.0)

    iota_n = lax.broadcasted_iota(jnp.int32, (n, 128), 0)
    big = jnp.int32(2**30)

    def round_(k, _):
        dd = d_ref[...]
        m = jnp.min(dd, axis=0, keepdims=True)
        am = jnp.min(jnp.where(dd == m, iota_n, big), axis=0, keepdims=True)
        out_ref[0, pl.ds(k, 1), :] = am
        d_ref[...] = jnp.where(iota_n == am, jnp.inf, dd)
        return 0

    lax.fori_loop(0, _K + 1, round_, 0)


def _topk_pallas(q_t, cands):
    B = q_t.shape[0]
    n = cands.shape[1]
    body = functools.partial(_topk_body, n, min(n, 2048))
    return pl.pallas_call(
        body,
        grid=(B, 8),
        in_specs=[
            pl.BlockSpec((1, 3, 128), lambda b, j: (b, 0, j)),
            pl.BlockSpec((1, n, 3), lambda b, j: (b, 0, 0)),
        ],
        out_specs=pl.BlockSpec((1, 40, 128), lambda b, j: (b, 0, j)),
        out_shape=jax.ShapeDtypeStruct((B, 40, 1024), jnp.int32),
        scratch_shapes=[pltpu.VMEM((n, 128), jnp.float32)],
    )(q_t, cands)


def _pdist2squared(x, y):
    xx = jnp.sum(x ** 2, axis=1)[:, :, None]
    yy = jnp.sum(y ** 2, axis=1)[:, None, :]
    dist = xx + yy - 2.0 * jnp.einsum('bdn,bdm->bnm', x, y)
    dist = jnp.nan_to_num(dist, nan=0.0)
    return jnp.clip(dist, 0.0, jnp.inf)


def _knn_ind(xyz2, xyz1, k):
    dist = _pdist2squared(jnp.transpose(xyz2, (0, 2, 1)),
                          jnp.transpose(xyz1, (0, 2, 1)))
    dist_t = jnp.transpose(dist, (0, 2, 1))
    _, idx = jax.lax.top_k(-dist_t, k + 1)
    return idx[:, :, 1:]


def kernel(points):
    B, N, _ = points.shape
    xyz_ind = _fps_pallas(points)
    xyz_query = jax.vmap(lambda p, i: p[i])(points, xyz_ind)

    q_t = jnp.transpose(xyz_query, (0, 2, 1))
    idx = _topk_pallas(q_t, points)
    neighbors_mid = jnp.transpose(idx[:, 1:_K + 1, :], (0, 2, 1))
    src_mid = neighbors_mid.reshape(B, -1)
    dst_mid = jnp.repeat(xyz_ind, _K, axis=1)
    d_mid = jax.vmap(lambda p, d, s: p[d] - p[s])(points, dst_mid, src_mid)

    idx2 = _topk_pallas(q_t, xyz_query)
    neighbors_out = jnp.transpose(idx2[:, 1:_K + 1, :], (0, 2, 1))
    src_out = neighbors_out.reshape(B, -1)
    dst_out = jnp.tile(jnp.repeat(jnp.arange(_NP, dtype=jnp.int32), _K)[None, :], (B, 1))
    d_out = jax.vmap(lambda p, d, s: p[d] - p[s])(xyz_query, dst_out, src_out)

    return (xyz_query, d_mid, d_out, xyz_ind, neighbors_mid, neighbors_out)


# speedup vs baseline: 4.5538x; 2.1975x over previous
import functools

import jax
import jax.numpy as jnp
from jax import lax
from jax.experimental import pallas as pl
from jax.experimental.pallas import tpu as pltpu
from jax.experimental.pallas import tpu_sc as plsc

_N = 16384
_NP = 1024
_K = 32
_R = 128
_C = 128


def _fps_body(px_ref, py_ref, pz_ref, out_ref, dists_ref):
    b = pl.program_id(0)
    px = px_ref[0]
    py = py_ref[0]
    pz = pz_ref[0]
    dists_ref[...] = jnp.full((_R, _C), jnp.inf, jnp.float32)
    out_ref[b, 0] = jnp.int32(0)

    lane_iota = lax.broadcasted_iota(jnp.int32, (1, _C), 1)
    row_iota = lax.broadcasted_iota(jnp.int32, (_R, _C), 0)
    col_iota = lax.broadcasted_iota(jnp.int32, (_R, _C), 1)
    lin_iota = row_iota * _C + col_iota

    def body(i, last):
        r = last // _C
        c = last % _C
        cm = lane_iota == c
        qx = jnp.sum(jnp.where(cm, px_ref[0, pl.ds(r, 1), :], 0.0))
        qy = jnp.sum(jnp.where(cm, py_ref[0, pl.ds(r, 1), :], 0.0))
        qz = jnp.sum(jnp.where(cm, pz_ref[0, pl.ds(r, 1), :], 0.0))
        dx = px - qx
        dy = py - qy
        dz = pz - qz
        d = dx * dx + dy * dy + dz * dz
        dd = jnp.minimum(dists_ref[...], d)
        dists_ref[...] = dd
        m = jnp.max(dd)
        nxt = jnp.min(jnp.where(dd == m, lin_iota, jnp.int32(2**30)))
        out_ref[b, i] = nxt
        return nxt

    lax.fori_loop(1, _NP, body, jnp.int32(0))


def _fps_pallas(points):
    px = points[:, :, 0].reshape(2, _R, _C)
    py = points[:, :, 1].reshape(2, _R, _C)
    pz = points[:, :, 2].reshape(2, _R, _C)
    return pl.pallas_call(
        _fps_body,
        grid=(2,),
        in_specs=[pl.BlockSpec((1, _R, _C), lambda b: (b, 0, 0))] * 3,
        out_specs=pl.BlockSpec(memory_space=pltpu.SMEM),
        out_shape=jax.ShapeDtypeStruct((2, _NP), jnp.int32),
        scratch_shapes=[pltpu.VMEM((_R, _C), jnp.float32)],
    )(px, py, pz)


def _dist_body(q_ref, ct_ref, o_ref):
    q = q_ref[0]
    qx = q[:, 0:1]
    qy = q[:, 1:2]
    qz = q[:, 2:3]
    cx = ct_ref[0, 0:1, :]
    cy = ct_ref[0, 1:2, :]
    cz = ct_ref[0, 2:3, :]
    qq = qx * qx + qy * qy + qz * qz
    cc = cx * cx + cy * cy + cz * cz
    f32 = jnp.float32
    bf = jnp.bfloat16
    qxb, qyb, qzb = (v.astype(bf).astype(f32) for v in (qx, qy, qz))
    cxb, cyb, czb = (v.astype(bf).astype(f32) for v in (cx, cy, cz))
    p0 = qxb * cxb
    p1 = qyb * cyb
    p2 = qzb * czb
    s = p0 + p1
    bb = s - p0
    e1 = (p0 - (s - bb)) + (p1 - bb)
    t = s + p2
    bb2 = t - s
    e2 = (s - (t - bb2)) + (p2 - bb2)
    e = t + (e1 + e2)
    d = (cc + qq) - 2.0 * e
    o_ref[0] = jnp.maximum(d, 0.0)


def _dist_pallas(q, ct, nq_blk, nc_blk):
    B, M, _ = q.shape
    N = ct.shape[2]
    grid = (B, M // nq_blk, N // nc_blk)
    return pl.pallas_call(
        _dist_body,
        grid=grid,
        in_specs=[
            pl.BlockSpec((1, nq_blk, 3), lambda b, j, i: (b, j, 0)),
            pl.BlockSpec((1, 3, nc_blk), lambda b, j, i: (b, 0, i)),
        ],
        out_specs=pl.BlockSpec((1, nq_blk, nc_blk), lambda b, j, i: (b, j, i)),
        out_shape=jax.ShapeDtypeStruct((B, M, N), jnp.float32),
    )(q, ct)


def _topk_body(n, nchunk, q_ref, c_ref, out_ref, d_ref):
    f32 = jnp.float32
    bf = jnp.bfloat16
    qx = q_ref[0, 0:1, :]
    qy = q_ref[0, 1:2, :]
    qz = q_ref[0, 2:3, :]
    qq = qx * qx + qy * qy + qz * qz
    qxb, qyb, qzb = (v.astype(bf).astype(f32) for v in (qx, qy, qz))
    for i in range(n // nchunk):
        sl = pl.ds(i * nchunk, nchunk)
        cx = c_ref[0, sl, 0:1]
        cy = c_ref[0, sl, 1:2]
        cz = c_ref[0, sl, 2:3]
        cc = cx * cx + cy * cy + cz * cz
        cxb, cyb, czb = (v.astype(bf).astype(f32) for v in (cx, cy, cz))
        p0 = cxb * qxb
        p1 = cyb * qyb
        p2 = czb * qzb
        s = p0 + p1
        bb = s - p0
        e1 = (p0 - (s - bb)) + (p1 - bb)
        t = s + p2
        bb2 = t - s
        e2 = (s - (t - bb2)) + (p2 - bb2)
        e = t + (e1 + e2)
        d_ref[sl, :] = jnp.maximum((cc + qq) - 2.0 * e, 0.0)

    iota_n = lax.broadcasted_iota(jnp.int32, (n, 128), 0)
    big = jnp.int32(2**30)

    def round_(k, _):
        dd = d_ref[...]
        m = jnp.min(dd, axis=0, keepdims=True)
        am = jnp.min(jnp.where(dd == m, iota_n, big), axis=0, keepdims=True)
        out_ref[0, pl.ds(k, 1), :] = am
        d_ref[...] = jnp.where(iota_n == am, jnp.inf, dd)
        return 0

    lax.fori_loop(0, _K + 1, round_, 0)


def _topk_pallas(q_t, cands):
    B = q_t.shape[0]
    n = cands.shape[1]
    body = functools.partial(_topk_body, n, min(n, 2048))
    return pl.pallas_call(
        body,
        grid=(B, 8),
        in_specs=[
            pl.BlockSpec((1, 3, 128), lambda b, j: (b, 0, j)),
            pl.BlockSpec((1, n, 3), lambda b, j: (b, 0, 0)),
        ],
        out_specs=pl.BlockSpec((1, 40, 128), lambda b, j: (b, 0, j)),
        out_shape=jax.ShapeDtypeStruct((B, 40, 1024), jnp.int32),
        scratch_shapes=[pltpu.VMEM((n, 128), jnp.float32)],
    )(q_t, cands)


_SC_MESH = plsc.VectorSubcoreMesh(core_axis_name="c", subcore_axis_name="s")


def _xq_gather_sc(points_flat, xyz_ind):
    rows = _NP // 16

    @functools.partial(
        pl.kernel, mesh=_SC_MESH,
        compiler_params=pltpu.CompilerParams(needs_layout_passes=False),
        out_type=jax.ShapeDtypeStruct((2, 16, 3, rows), jnp.float32),
        scratch_types=[
            pltpu.VMEM((_N * 3,), jnp.float32),
            pltpu.VMEM((rows,), jnp.int32),
            pltpu.VMEM((3, rows), jnp.float32),
        ],
    )
    def k(pts_hbm, idx_hbm, out_hbm, pts_v, idx_v, out_v):
        c = lax.axis_index("c")
        s = lax.axis_index("s")
        pltpu.sync_copy(pts_hbm.at[c], pts_v)
        pltpu.sync_copy(idx_hbm.at[c, pl.ds(s * rows, rows)], idx_v)

        def body(j, _):
            sl = pl.ds(j * 16, 16)
            vi = idx_v[sl] * 3
            for d in range(3):
                out_v[d, sl] = plsc.load_gather(pts_v, [vi + d])
            return 0

        lax.fori_loop(0, rows // 16, body, 0)
        pltpu.sync_copy(out_v, out_hbm.at[c, s])

    packed = k(points_flat, xyz_ind)
    return jnp.transpose(packed, (0, 1, 3, 2)).reshape(2, _NP, 3)


def _edges_sc(points_flat, xq_flat, nbr_mid_flat, nbr_out_flat):
    E = (_NP * _K) // 16

    @functools.partial(
        pl.kernel, mesh=_SC_MESH,
        compiler_params=pltpu.CompilerParams(needs_layout_passes=False),
        out_type=(jax.ShapeDtypeStruct((2, 16, 3, E), jnp.float32),
                  jax.ShapeDtypeStruct((2, 16, 3, E), jnp.float32)),
        scratch_types=[
            pltpu.VMEM((_N * 3,), jnp.float32),
            pltpu.VMEM((_NP * 3,), jnp.float32),
            pltpu.VMEM((E,), jnp.int32),
            pltpu.VMEM((E,), jnp.int32),
            pltpu.VMEM((3, E), jnp.float32),
            pltpu.VMEM((3, E), jnp.float32),
        ],
    )
    def k(pts_hbm, xq_hbm, nm_hbm, no_hbm, dm_hbm, do_hbm,
          pts_v, xq_v, nm_v, no_v, dm_v, do_v):
        c = lax.axis_index("c")
        s = lax.axis_index("s")
        pltpu.sync_copy(pts_hbm.at[c], pts_v)
        pltpu.sync_copy(xq_hbm.at[c], xq_v)
        pltpu.sync_copy(nm_hbm.at[c, pl.ds(s * E, E)], nm_v)
        pltpu.sync_copy(no_hbm.at[c, pl.ds(s * E, E)], no_v)
        base = s * E
        lane = jax.lax.iota(jnp.int32, 16)

        def body(j, _):
            sl = pl.ds(j * 16, 16)
            qi = ((base + j * 16 + lane) >> 5) * 3
            si_m = nm_v[sl] * 3
            si_o = no_v[sl] * 3
            for d in range(3):
                dst = plsc.load_gather(xq_v, [qi + d])
                dm_v[d, sl] = dst - plsc.load_gather(pts_v, [si_m + d])
                do_v[d, sl] = dst - plsc.load_gather(xq_v, [si_o + d])
            return 0

        lax.fori_loop(0, E // 16, body, 0)
        pltpu.sync_copy(dm_v, dm_hbm.at[c, s])
        pltpu.sync_copy(do_v, do_hbm.at[c, s])

    dm, do = k(points_flat, xq_flat, nbr_mid_flat, nbr_out_flat)
    d_mid = jnp.transpose(dm, (0, 1, 3, 2)).reshape(2, _NP * _K, 3)
    d_out = jnp.transpose(do, (0, 1, 3, 2)).reshape(2, _NP * _K, 3)
    return d_mid, d_out


def _pdist2squared(x, y):
    xx = jnp.sum(x ** 2, axis=1)[:, :, None]
    yy = jnp.sum(y ** 2, axis=1)[:, None, :]
    dist = xx + yy - 2.0 * jnp.einsum('bdn,bdm->bnm', x, y)
    dist = jnp.nan_to_num(dist, nan=0.0)
    return jnp.clip(dist, 0.0, jnp.inf)


def _knn_ind(xyz2, xyz1, k):
    dist = _pdist2squared(jnp.transpose(xyz2, (0, 2, 1)),
                          jnp.transpose(xyz1, (0, 2, 1)))
    dist_t = jnp.transpose(dist, (0, 2, 1))
    _, idx = jax.lax.top_k(-dist_t, k + 1)
    return idx[:, :, 1:]


def kernel(points):
    B, N, _ = points.shape
    xyz_ind = _fps_pallas(points)
    points_flat = points.reshape(B, N * 3)
    xyz_query = _xq_gather_sc(points_flat, xyz_ind)

    q_t = jnp.transpose(xyz_query, (0, 2, 1))
    idx = _topk_pallas(q_t, points)
    neighbors_mid = jnp.transpose(idx[:, 1:_K + 1, :], (0, 2, 1))

    idx2 = _topk_pallas(q_t, xyz_query)
    neighbors_out = jnp.transpose(idx2[:, 1:_K + 1, :], (0, 2, 1))

    d_mid, d_out = _edges_sc(points_flat, xyz_query.reshape(B, _NP * 3),
                             neighbors_mid.reshape(B, -1),
                             neighbors_out.reshape(B, -1))

    return (xyz_query, d_mid, d_out, xyz_ind, neighbors_mid, neighbors_out)
